# baseline (speedup 1.0000x reference)
import functools

import jax
import jax.numpy as jnp
from jax import lax
from jax.experimental import pallas as pl
from jax.experimental.pallas import tpu as pltpu
from jax.experimental.pallas import tpu_sc as plsc

N = 1024
F = 64
B = 32
CH = 32
NC, NS = 2, 16
EU = 16
ZU = 8


_GDN = lax.GatherDimensionNumbers(
    offset_dims=(), collapsed_slice_dims=(0,), start_index_map=(0,))


def _lane_bcast(vec, j):
    idx = jnp.full((16,), j, jnp.int32)
    return lax.gather(vec, idx[:, None], _GDN, slice_sizes=(1,),
                      mode=lax.GatherScatterMode.PROMISE_IN_BOUNDS)


def _edge_sweep(rbuf, cbuf, vbuf, nnzp, src, dst, double):
    iota = lax.iota(jnp.int32, 16)

    @plsc.parallel_loop(0, nnzp, EU, unroll=2)
    def _(base):
        rv = rbuf[pl.ds(base, EU)]
        cv = cbuf[pl.ds(base, EU)]
        vv = vbuf[pl.ds(base, EU)]
        if double:
            vv = vv + vv
        for j in range(EU):
            cb = _lane_bcast(cv, j)
            rb = _lane_bcast(rv, j)
            vb = _lane_bcast(vv, j)
            for blk in range(CH // 16):
                off = iota + (blk * 16)
                x = plsc.load_gather(src, [cb, off])
                plsc.addupdate_scatter(dst, [rb, off], x * vb)


def _fill_zero(buf):
    z = jnp.zeros((16,), jnp.float32)

    @plsc.parallel_loop(0, N, ZU)
    def _(r0):
        for j in range(ZU):
            for blk in range(CH // 16):
                buf[r0 + j, pl.ds(blk * 16, 16)] = z


def _negate(buf):
    @plsc.parallel_loop(0, N, ZU)
    def _(r0):
        for j in range(ZU):
            for blk in range(CH // 16):
                v = buf[r0 + j, pl.ds(blk * 16, 16)]
                buf[r0 + j, pl.ds(blk * 16, 16)] = -v


def _make_diffusion(nnz1p, nnz2p, nsrc):
    nnzm = max(nnz1p, nnz2p)
    cw = nsrc * F
    nfs = cw // CH

    mesh = plsc.VectorSubcoreMesh(
        core_axis_name="c", subcore_axis_name="s",
        num_cores=NC, num_subcores=NS)

    out = jax.ShapeDtypeStruct((B, N, cw), jnp.float32)

    @functools.partial(
        pl.kernel,
        out_type=(out, out, out, out),
        mesh=mesh,
        scratch_types=[
            pltpu.VMEM((nnzm,), jnp.int32),
            pltpu.VMEM((nnzm,), jnp.int32),
            pltpu.VMEM((nnzm,), jnp.float32),
            pltpu.VMEM((N, CH), jnp.float32),
            pltpu.VMEM((N, CH), jnp.float32),
        ],
        compiler_params=pltpu.CompilerParams(
            use_tc_tiling_on_sc=False, needs_layout_passes=False),
    )
    def diffusion(r1, c1, v1, r2, c2, v2, src, y11, y12, y21, y22,
                  rbuf, cbuf, vbuf, xbuf, ybuf):
        b = lax.axis_index("s") * NC + lax.axis_index("c")

        for rh, ch_, vh, nnzp, o1, o2 in [
                (r1, c1, v1, nnz1p, y11, y12),
                (r2, c2, v2, nnz2p, y21, y22)]:
            pltpu.sync_copy(rh, rbuf.at[pl.ds(0, nnzp)])
            pltpu.sync_copy(ch_, cbuf.at[pl.ds(0, nnzp)])
            pltpu.sync_copy(vh, vbuf.at[pl.ds(0, nnzp)])

            def chunk_body(fs, carry):
                col0 = fs * CH
                pltpu.sync_copy(src.at[b, :, pl.ds(col0, CH)], xbuf)
                _fill_zero(ybuf)
                _edge_sweep(rbuf, cbuf, vbuf, nnzp, xbuf, ybuf, False)
                pltpu.sync_copy(ybuf, o1.at[b, :, pl.ds(col0, CH)])
                _negate(xbuf)
                _edge_sweep(rbuf, cbuf, vbuf, nnzp, ybuf, xbuf, True)
                pltpu.sync_copy(xbuf, o2.at[b, :, pl.ds(col0, CH)])
                return carry

            lax.fori_loop(0, nfs, chunk_body, 0)

    return diffusion


def _tc_gate1(inp, hxb, a1, a2, b1, b2, ws, bru, hp_out, u_out):
    x = inp[0]
    h = hxb[0]
    w = ws[...]
    acc = (
        jnp.dot(x, w[0], preferred_element_type=jnp.float32)
        + jnp.dot(h, w[1], preferred_element_type=jnp.float32)
        + jnp.dot(a1[0, :, :F], w[2], preferred_element_type=jnp.float32)
        + jnp.dot(a1[0, :, F:], w[3], preferred_element_type=jnp.float32)
        + jnp.dot(a2[0, :, :F], w[4], preferred_element_type=jnp.float32)
        + jnp.dot(a2[0, :, F:], w[5], preferred_element_type=jnp.float32)
        + jnp.dot(b1[0, :, :F], w[6], preferred_element_type=jnp.float32)
        + jnp.dot(b1[0, :, F:], w[7], preferred_element_type=jnp.float32)
        + jnp.dot(b2[0, :, :F], w[8], preferred_element_type=jnp.float32)
        + jnp.dot(b2[0, :, F:], w[9], preferred_element_type=jnp.float32)
        + bru[...]
    )
    val = jax.nn.sigmoid(acc)
    r = val[:, :F]
    u = val[:, F:]
    hp_out[0] = r * h
    u_out[0] = u


def _tc_gate2(inp, hxb, hp, c1, c2, d1, d2, a1, a2, b1, b2, ws, bc, ub,
              out):
    x = inp[0]
    h = hxb[0]
    w = ws[...]
    acc = (
        jnp.dot(x, w[0], preferred_element_type=jnp.float32)
        + jnp.dot(hp[0], w[1], preferred_element_type=jnp.float32)
        + jnp.dot(a1[0, :, :F], w[2], preferred_element_type=jnp.float32)
        + jnp.dot(c1[0], w[3], preferred_element_type=jnp.float32)
        + jnp.dot(a2[0, :, :F], w[4], preferred_element_type=jnp.float32)
        + jnp.dot(c2[0], w[5], preferred_element_type=jnp.float32)
        + jnp.dot(b1[0, :, :F], w[6], preferred_element_type=jnp.float32)
        + jnp.dot(d1[0], w[7], preferred_element_type=jnp.float32)
        + jnp.dot(b2[0, :, :F], w[8], preferred_element_type=jnp.float32)
        + jnp.dot(d2[0], w[9], preferred_element_type=jnp.float32)
        + bc[...]
    )
    c = jnp.tanh(acc)
    u = ub[0]
    out[0] = u * h + (1.0 - u) * c


def _pad_edges(r, c, v, mult):
    nnz = r.shape[0]
    pad = (-nnz) % mult
    if pad:
        r = jnp.pad(r, (0, pad))
        c = jnp.pad(c, (0, pad))
        v = jnp.pad(v, (0, pad))
    return r, c, v, nnz + pad


def _split_w(w, num_m):
    parts = []
    for m in range(num_m):
        wm = w[m::num_m]
        parts.append(wm[:F])
        parts.append(wm[F:])
    return jnp.stack(parts)


def kernel(inputs, hx, s1_row, s1_col, s1_val, s2_row, s2_col, s2_val,
           W_ru, b_ru, W_c, b_c):
    xin = inputs.reshape(B, N, F)
    h = hx.reshape(B, N, F)

    r1, c1, v1, nnz1p = _pad_edges(s1_row, s1_col, s1_val, EU)
    r2, c2, v2, nnz2p = _pad_edges(s2_row, s2_col, s2_val, EU)

    diff1 = _make_diffusion(nnz1p, nnz2p, 2)
    x0 = jnp.concatenate([xin, h], axis=2)
    a1, a2, b1, b2 = diff1(r1, c1, v1, r2, c2, v2, x0)

    ws_ru = _split_w(W_ru, 5)
    ws_c = _split_w(W_c, 5)
    bru2 = b_ru.reshape(1, 2 * F)
    bc2 = b_c.reshape(1, F)

    spec_bn = lambda w: pl.BlockSpec((1, N, w), lambda i: (i, 0, 0))
    full = lambda a: pl.BlockSpec(a.shape, lambda i: (0,) * a.ndim)
    hp, u = pl.pallas_call(
        _tc_gate1,
        grid=(B,),
        in_specs=[spec_bn(F), spec_bn(F), spec_bn(2 * F), spec_bn(2 * F),
                  spec_bn(2 * F), spec_bn(2 * F), full(ws_ru),
                  pl.BlockSpec((1, 2 * F), lambda i: (0, 0))],
        out_specs=[spec_bn(F), spec_bn(F)],
        out_shape=[jax.ShapeDtypeStruct((B, N, F), jnp.float32),
                   jax.ShapeDtypeStruct((B, N, F), jnp.float32)],
    )(xin, h, a1, a2, b1, b2, ws_ru, bru2)

    diff2 = _make_diffusion(nnz1p, nnz2p, 1)
    cc1, cc2, dd1, dd2 = diff2(r1, c1, v1, r2, c2, v2, hp)

    new_state = pl.pallas_call(
        _tc_gate2,
        grid=(B,),
        in_specs=[spec_bn(F), spec_bn(F), spec_bn(F), spec_bn(F),
                  spec_bn(F), spec_bn(F), spec_bn(F), spec_bn(2 * F),
                  spec_bn(2 * F), spec_bn(2 * F), spec_bn(2 * F),
                  full(ws_c), pl.BlockSpec((1, F), lambda i: (0, 0)),
                  spec_bn(F)],
        out_specs=spec_bn(F),
        out_shape=jax.ShapeDtypeStruct((B, N, F), jnp.float32),
    )(xin, h, hp, cc1, cc2, dd1, dd2, a1, a2, b1, b2, ws_c, bc2, u)

    return new_state.reshape(B, N * F)


# speedup vs baseline: 1.1571x; 1.1571x over previous
import functools

import jax
import jax.numpy as jnp
from jax import lax
from jax.experimental import pallas as pl
from jax.experimental.pallas import tpu as pltpu
from jax.experimental.pallas import tpu_sc as plsc

N = 1024
F = 64
B = 32
CH = 32
NC, NS = 2, 16
EU = 16
ZU = 8

_GDN = lax.GatherDimensionNumbers(
    offset_dims=(), collapsed_slice_dims=(0,), start_index_map=(0,))


def _lane_bcast(vec, j):
    idx = jnp.full((16,), j, jnp.int32)
    return lax.gather(vec, idx[:, None], _GDN, slice_sizes=(1,),
                      mode=lax.GatherScatterMode.PROMISE_IN_BOUNDS)


def _edge_sweep(pbuf, vbuf, nnzp, src, dst, double):
    iota = lax.iota(jnp.int32, 16)

    @plsc.parallel_loop(0, nnzp, EU)
    def _(base):
        pv = pbuf[pl.ds(base, EU)]
        vv = vbuf[pl.ds(base, EU)]
        if double:
            vv = vv + vv
        for j in range(EU):
            pb = _lane_bcast(pv, j)
            vb = _lane_bcast(vv, j)
            coff = jnp.bitwise_and(pb, 0xFFFF)
            roff = lax.shift_right_logical(pb, 16)
            for blk in range(CH // 16):
                off = iota + (blk * 16)
                x = plsc.load_gather(src, [coff + off])
                plsc.addupdate_scatter(dst, [roff + off], x * vb)


def _fill_zero(buf):
    z = jnp.zeros((16,), jnp.float32)

    @plsc.parallel_loop(0, N * CH, 16 * ZU)
    def _(r0):
        for j in range(ZU):
            buf[pl.ds(r0 + j * 16, 16)] = z


def _negate(buf):
    @plsc.parallel_loop(0, N * CH, 16 * ZU)
    def _(r0):
        for j in range(ZU):
            buf[pl.ds(r0 + j * 16, 16)] = -buf[pl.ds(r0 + j * 16, 16)]


def _make_diffusion(nnz1p, nnz2p, nfs):
    nnzm = max(nnz1p, nnz2p)

    mesh = plsc.VectorSubcoreMesh(
        core_axis_name="c", subcore_axis_name="s",
        num_cores=NC, num_subcores=NS)

    out = jax.ShapeDtypeStruct((B, nfs, N * CH), jnp.float32)

    @functools.partial(
        pl.kernel,
        out_type=(out, out, out, out),
        mesh=mesh,
        scratch_types=[
            pltpu.VMEM((nnzm,), jnp.int32),
            pltpu.VMEM((nnzm,), jnp.float32),
            pltpu.VMEM((N * CH,), jnp.float32),
            pltpu.VMEM((N * CH,), jnp.float32),
        ],
        compiler_params=pltpu.CompilerParams(
            use_tc_tiling_on_sc=False, needs_layout_passes=False),
    )
    def diffusion(p1, v1, p2, v2, src, y11, y12, y21, y22,
                  pbuf, vbuf, xbuf, ybuf):
        b = lax.axis_index("s") * NC + lax.axis_index("c")

        for ph, vh, nnzp, o1, o2 in [
                (p1, v1, nnz1p, y11, y12),
                (p2, v2, nnz2p, y21, y22)]:
            pltpu.sync_copy(ph, pbuf.at[pl.ds(0, nnzp)])
            pltpu.sync_copy(vh, vbuf.at[pl.ds(0, nnzp)])

            def chunk_body(fs, carry):
                pltpu.sync_copy(src.at[b, fs], xbuf)
                _fill_zero(ybuf)
                _edge_sweep(pbuf, vbuf, nnzp, xbuf, ybuf, False)
                pltpu.sync_copy(ybuf, o1.at[b, fs])
                _negate(xbuf)
                _edge_sweep(pbuf, vbuf, nnzp, ybuf, xbuf, True)
                pltpu.sync_copy(xbuf, o2.at[b, fs])
                return carry

            lax.fori_loop(0, nfs, chunk_body, 0)

    return diffusion


def _tc_gate1(x0b, a1, a2, b1, b2, ws, bru, hp_out, u_out):
    w = ws[...]
    acc = bru[...]
    for ai, arr in enumerate((x0b, a1, a2, b1, b2)):
        for fs in range(4):
            acc = acc + jnp.dot(arr[0, fs], w[ai * 4 + fs],
                                preferred_element_type=jnp.float32)
    val = jax.nn.sigmoid(acc)
    r = val[:, :F]
    u = val[:, F:]
    h = jnp.concatenate([x0b[0, 2], x0b[0, 3]], axis=1)
    hp = r * h
    hp_out[0, 0] = hp[:, :CH]
    hp_out[0, 1] = hp[:, CH:]
    u_out[0] = u


def _tc_gate2(x0b, hpb, a1, c1, a2, c2, b1, d1, b2, d2, hxb, ub, ws, bc,
              out):
    w = ws[...]
    acc = bc[...]
    for ai, (ina, sta) in enumerate(
            ((x0b, hpb), (a1, c1), (a2, c2), (b1, d1), (b2, d2))):
        acc = acc + jnp.dot(ina[0, 0], w[ai * 4],
                            preferred_element_type=jnp.float32)
        acc = acc + jnp.dot(ina[0, 1], w[ai * 4 + 1],
                            preferred_element_type=jnp.float32)
        acc = acc + jnp.dot(sta[0, 0], w[ai * 4 + 2],
                            preferred_element_type=jnp.float32)
        acc = acc + jnp.dot(sta[0, 1], w[ai * 4 + 3],
                            preferred_element_type=jnp.float32)
    c = jnp.tanh(acc)
    u = ub[0]
    out[0] = u * hxb[0] + (1.0 - u) * c


def _pack_edges(r, c, v):
    nnz = r.shape[0]
    pad = (-nnz) % EU
    if pad:
        r = jnp.pad(r, (0, pad))
        c = jnp.pad(c, (0, pad))
        v = jnp.pad(v, (0, pad))
    pk = jnp.left_shift(r, 21) | jnp.left_shift(c, 5)
    return pk, v, nnz + pad


def _split_w(w, num_m):
    parts = []
    for m in range(num_m):
        wm = w[m::num_m]
        for q in range(4):
            parts.append(wm[q * CH:(q + 1) * CH])
    return jnp.stack(parts)


def kernel(inputs, hx, s1_row, s1_col, s1_val, s2_row, s2_col, s2_val,
           W_ru, b_ru, W_c, b_c):
    xin = inputs.reshape(B, N, F)
    h = hx.reshape(B, N, F)

    p1, v1, nnz1p = _pack_edges(s1_row, s1_col, s1_val)
    p2, v2, nnz2p = _pack_edges(s2_row, s2_col, s2_val)

    x0c = (jnp.concatenate([xin, h], axis=2)
           .reshape(B, N, 4, CH).transpose(0, 2, 1, 3))
    x0f = x0c.reshape(B, 4, N * CH)

    diff1 = _make_diffusion(nnz1p, nnz2p, 4)
    a1, a2, b1, b2 = diff1(p1, v1, p2, v2, x0f)

    ws_ru = _split_w(W_ru, 5)
    ws_c = _split_w(W_c, 5)
    bru2 = b_ru.reshape(1, 2 * F)
    bc2 = b_c.reshape(1, F)

    c4 = lambda: pl.BlockSpec((1, 4, N, CH), lambda i: (i, 0, 0, 0))
    c2s = lambda: pl.BlockSpec((1, 2, N, CH), lambda i: (i, 0, 0, 0))
    bn = lambda wdt: pl.BlockSpec((1, N, wdt), lambda i: (i, 0, 0))
    full = lambda a: pl.BlockSpec(a.shape, lambda i: (0,) * a.ndim)
    v4 = lambda a: a.reshape(B, 4, N, CH)
    v2s = lambda a: a.reshape(B, 2, N, CH)

    hp, u = pl.pallas_call(
        _tc_gate1,
        grid=(B,),
        in_specs=[c4(), c4(), c4(), c4(), c4(), full(ws_ru),
                  pl.BlockSpec((1, 2 * F), lambda i: (0, 0))],
        out_specs=[c2s(), bn(F)],
        out_shape=[jax.ShapeDtypeStruct((B, 2, N, CH), jnp.float32),
                   jax.ShapeDtypeStruct((B, N, F), jnp.float32)],
    )(v4(x0f), v4(a1), v4(a2), v4(b1), v4(b2), ws_ru, bru2)

    diff2 = _make_diffusion(nnz1p, nnz2p, 2)
    hpf = hp.reshape(B, 2, N * CH)
    cc1, cc2, dd1, dd2 = diff2(p1, v1, p2, v2, hpf)

    new_state = pl.pallas_call(
        _tc_gate2,
        grid=(B,),
        in_specs=[c4(), c2s(), c4(), c2s(), c4(), c2s(), c4(), c2s(),
                  c4(), c2s(), bn(F), bn(F), full(ws_c),
                  pl.BlockSpec((1, F), lambda i: (0, 0))],
        out_specs=bn(F),
        out_shape=jax.ShapeDtypeStruct((B, N, F), jnp.float32),
    )(v4(x0f), hp, v4(a1), v2s(cc1), v4(a2), v2s(cc2), v4(b1), v2s(dd1),
      v4(b2), v2s(dd2), h, u, ws_c, bc2)

    return new_state.reshape(B, N * F)


# speedup vs baseline: 2.9270x; 2.5295x over previous
import functools

import jax
import jax.numpy as jnp
from jax import lax
from jax.experimental import pallas as pl
from jax.experimental.pallas import tpu as pltpu
from jax.experimental.pallas import tpu_sc as plsc

N = 1024
F = 64
B = 32
CH = 32
NC, NS = 2, 16
EU = 16
ZU = 8

_GDN = lax.GatherDimensionNumbers(
    offset_dims=(), collapsed_slice_dims=(0,), start_index_map=(0,))


def _lane_bcast(vec, j):
    idx = jnp.full((16,), j, jnp.int32)
    return lax.gather(vec, idx[:, None], _GDN, slice_sizes=(1,),
                      mode=lax.GatherScatterMode.PROMISE_IN_BOUNDS)


def _edge_sweep(rbuf, cbuf, vbuf, nnzp, src, dst, double):
    iota = lax.iota(jnp.int32, 16)

    @plsc.parallel_loop(0, nnzp, EU)
    def _(base):
        rv = rbuf[pl.ds(base, EU)]
        cv = cbuf[pl.ds(base, EU)]
        vv = vbuf[pl.ds(base, EU)]
        if double:
            vv = vv + vv
        for j in range(EU):
            cb = _lane_bcast(cv, j)
            rb = _lane_bcast(rv, j)
            vb = _lane_bcast(vv, j)
            for blk in range(CH // 16):
                off = iota + (blk * 16)
                x = plsc.load_gather(src, [cb, off])
                plsc.addupdate_scatter(dst, [rb, off], x * vb)


def _fill_zero(buf):
    z = jnp.zeros((16,), jnp.float32)

    @plsc.parallel_loop(0, N, ZU)
    def _(r0):
        for j in range(ZU):
            for blk in range(CH // 16):
                buf[r0 + j, pl.ds(blk * 16, 16)] = z


def _negate(buf):
    @plsc.parallel_loop(0, N, ZU)
    def _(r0):
        for j in range(ZU):
            for blk in range(CH // 16):
                v = buf[r0 + j, pl.ds(blk * 16, 16)]
                buf[r0 + j, pl.ds(blk * 16, 16)] = -v


def _make_diffusion(nnz1p, nnz2p, nsrc):
    nnzm = max(nnz1p, nnz2p)
    cw = nsrc * F
    nfs = cw // CH

    mesh = plsc.VectorSubcoreMesh(
        core_axis_name="c", subcore_axis_name="s",
        num_cores=NC, num_subcores=NS)

    out = jax.ShapeDtypeStruct((B, N, cw), jnp.float32)

    @functools.partial(
        pl.kernel,
        out_type=(out, out, out, out),
        mesh=mesh,
        scratch_types=[
            pltpu.VMEM((nnzm,), jnp.int32),
            pltpu.VMEM((nnzm,), jnp.int32),
            pltpu.VMEM((nnzm,), jnp.float32),
            pltpu.VMEM((N, CH), jnp.float32),
            pltpu.VMEM((N, CH), jnp.float32),
        ],
        compiler_params=pltpu.CompilerParams(
            use_tc_tiling_on_sc=False, needs_layout_passes=False),
    )
    def diffusion(r1, c1, v1, r2, c2, v2, src, y11, y12, y21, y22,
                  rbuf, cbuf, vbuf, xbuf, ybuf):
        b = lax.axis_index("s") * NC + lax.axis_index("c")

        for rh, ch_, vh, nnzp, o1, o2 in [
                (r1, c1, v1, nnz1p, y11, y12),
                (r2, c2, v2, nnz2p, y21, y22)]:
            pltpu.sync_copy(rh, rbuf.at[pl.ds(0, nnzp)])
            pltpu.sync_copy(ch_, cbuf.at[pl.ds(0, nnzp)])
            pltpu.sync_copy(vh, vbuf.at[pl.ds(0, nnzp)])

            def chunk_body(fs, carry):
                col0 = fs * CH
                pltpu.sync_copy(src.at[b, :, pl.ds(col0, CH)], xbuf)
                _fill_zero(ybuf)
                _edge_sweep(rbuf, cbuf, vbuf, nnzp, xbuf, ybuf, False)
                pltpu.sync_copy(ybuf, o1.at[b, :, pl.ds(col0, CH)])
                _negate(xbuf)
                _edge_sweep(rbuf, cbuf, vbuf, nnzp, ybuf, xbuf, True)
                pltpu.sync_copy(xbuf, o2.at[b, :, pl.ds(col0, CH)])
                return carry

            lax.fori_loop(0, nfs, chunk_body, 0)

    return diffusion


def _tc_gate1(inp, hxb, a1, a2, b1, b2, ws, bru, hp_out, u_out):
    x = inp[0]
    h = hxb[0]
    w = ws[...]
    acc = (
        jnp.dot(x, w[0], preferred_element_type=jnp.float32)
        + jnp.dot(h, w[1], preferred_element_type=jnp.float32)
        + jnp.dot(a1[0, :, :F], w[2], preferred_element_type=jnp.float32)
        + jnp.dot(a1[0, :, F:], w[3], preferred_element_type=jnp.float32)
        + jnp.dot(a2[0, :, :F], w[4], preferred_element_type=jnp.float32)
        + jnp.dot(a2[0, :, F:], w[5], preferred_element_type=jnp.float32)
        + jnp.dot(b1[0, :, :F], w[6], preferred_element_type=jnp.float32)
        + jnp.dot(b1[0, :, F:], w[7], preferred_element_type=jnp.float32)
        + jnp.dot(b2[0, :, :F], w[8], preferred_element_type=jnp.float32)
        + jnp.dot(b2[0, :, F:], w[9], preferred_element_type=jnp.float32)
        + bru[...]
    )
    val = jax.nn.sigmoid(acc)
    r = val[:, :F]
    u = val[:, F:]
    hp_out[0] = r * h
    u_out[0] = u


def _tc_gate2(inp, hxb, hp, c1, c2, d1, d2, a1, a2, b1, b2, ws, bc, ub,
              out):
    x = inp[0]
    h = hxb[0]
    w = ws[...]
    acc = (
        jnp.dot(x, w[0], preferred_element_type=jnp.float32)
        + jnp.dot(hp[0], w[1], preferred_element_type=jnp.float32)
        + jnp.dot(a1[0, :, :F], w[2], preferred_element_type=jnp.float32)
        + jnp.dot(c1[0], w[3], preferred_element_type=jnp.float32)
        + jnp.dot(a2[0, :, :F], w[4], preferred_element_type=jnp.float32)
        + jnp.dot(c2[0], w[5], preferred_element_type=jnp.float32)
        + jnp.dot(b1[0, :, :F], w[6], preferred_element_type=jnp.float32)
        + jnp.dot(d1[0], w[7], preferred_element_type=jnp.float32)
        + jnp.dot(b2[0, :, :F], w[8], preferred_element_type=jnp.float32)
        + jnp.dot(d2[0], w[9], preferred_element_type=jnp.float32)
        + bc[...]
    )
    c = jnp.tanh(acc)
    u = ub[0]
    out[0] = u * h + (1.0 - u) * c


def _pad_edges(r, c, v, mult):
    nnz = r.shape[0]
    pad = (-nnz) % mult
    if pad:
        r = jnp.pad(r, (0, pad))
        c = jnp.pad(c, (0, pad))
        v = jnp.pad(v, (0, pad))
    return r, c, v, nnz + pad


def _split_w(w, num_m):
    parts = []
    for m in range(num_m):
        wm = w[m::num_m]
        parts.append(wm[:F])
        parts.append(wm[F:])
    return jnp.stack(parts)


def kernel(inputs, hx, s1_row, s1_col, s1_val, s2_row, s2_col, s2_val,
           W_ru, b_ru, W_c, b_c):
    xin = inputs.reshape(B, N, F)
    h = hx.reshape(B, N, F)

    r1, c1, v1, nnz1p = _pad_edges(s1_row, s1_col, s1_val, EU)
    r2, c2, v2, nnz2p = _pad_edges(s2_row, s2_col, s2_val, EU)

    diff1 = _make_diffusion(nnz1p, nnz2p, 2)
    x0 = jnp.concatenate([xin, h], axis=2)
    a1, a2, b1, b2 = diff1(r1, c1, v1, r2, c2, v2, x0)

    ws_ru = _split_w(W_ru, 5)
    ws_c = _split_w(W_c, 5)
    bru2 = b_ru.reshape(1, 2 * F)
    bc2 = b_c.reshape(1, F)

    spec_bn = lambda w: pl.BlockSpec((1, N, w), lambda i: (i, 0, 0))
    full = lambda a: pl.BlockSpec(a.shape, lambda i: (0,) * a.ndim)
    hp, u = pl.pallas_call(
        _tc_gate1,
        grid=(B,),
        in_specs=[spec_bn(F), spec_bn(F), spec_bn(2 * F), spec_bn(2 * F),
                  spec_bn(2 * F), spec_bn(2 * F), full(ws_ru),
                  pl.BlockSpec((1, 2 * F), lambda i: (0, 0))],
        out_specs=[spec_bn(F), spec_bn(F)],
        out_shape=[jax.ShapeDtypeStruct((B, N, F), jnp.float32),
                   jax.ShapeDtypeStruct((B, N, F), jnp.float32)],
    )(xin, h, a1, a2, b1, b2, ws_ru, bru2)

    diff2 = _make_diffusion(nnz1p, nnz2p, 1)
    cc1, cc2, dd1, dd2 = diff2(r1, c1, v1, r2, c2, v2, hp)

    new_state = pl.pallas_call(
        _tc_gate2,
        grid=(B,),
        in_specs=[spec_bn(F), spec_bn(F), spec_bn(F), spec_bn(F),
                  spec_bn(F), spec_bn(F), spec_bn(F), spec_bn(2 * F),
                  spec_bn(2 * F), spec_bn(2 * F), spec_bn(2 * F),
                  full(ws_c), pl.BlockSpec((1, F), lambda i: (0, 0)),
                  spec_bn(F)],
        out_specs=spec_bn(F),
        out_shape=jax.ShapeDtypeStruct((B, N, F), jnp.float32),
    )(xin, h, hp, cc1, cc2, dd1, dd2, a1, a2, b1, b2, ws_c, bc2, u)

    return new_state.reshape(B, N * F)
